# Initial kernel scaffold; baseline (speedup 1.0000x reference)
import jax
import jax.numpy as jnp
from jax.experimental import pallas as pl


def kernel(x):
    raise NotImplementedError("write your pallas kernel here")


# speedup vs baseline: 3.3627x; 3.3627x over previous
import numpy as np
import jax
import jax.numpy as jnp
from jax.experimental import pallas as pl
from jax.experimental.pallas import tpu as pltpu


def _dct_mat(N):
    n = np.arange(N, dtype=np.float64)
    k = np.arange(N, dtype=np.float64)[:, None]
    M = np.cos(np.pi * (n + 0.5) * k / N)
    scale = np.where(k == 0, np.sqrt(1.0 / N), np.sqrt(2.0 / N))
    return M * scale


_A_NP = (_dct_mat(32).T @ _dct_mat(64)[:32, :]).astype(np.float32)


def _pool_body(a_ref, x_ref, o_ref):
    A = a_ref[...]
    xb = x_ref[0][:, :, :32]
    t = jnp.dot(A, xb.reshape(64, 64 * 32),
                preferred_element_type=jnp.float32).reshape(32, 64, 32)
    tt = t.transpose(1, 0, 2).reshape(64, 32 * 32)
    o = jnp.dot(A, tt, preferred_element_type=jnp.float32).reshape(32, 32, 32)
    o_ref[0] = o.transpose(1, 0, 2)


def kernel(x):
    B, C, D, H, W = x.shape
    BC = B * C
    xr = x.reshape(BC, D, H, W)
    A = jnp.asarray(_A_NP)

    out = pl.pallas_call(
        _pool_body,
        grid=(BC,),
        in_specs=[
            pl.BlockSpec((32, 64), lambda i: (0, 0)),
            pl.BlockSpec((1, 64, 64, 64), lambda i: (i, 0, 0, 0)),
        ],
        out_specs=pl.BlockSpec((1, 32, 32, 32), lambda i: (i, 0, 0, 0)),
        out_shape=jax.ShapeDtypeStruct((BC, 32, 32, 32), jnp.float32),
        compiler_params=pltpu.CompilerParams(
            dimension_semantics=("parallel",),
        ),
    )(A, xr)
    return out.reshape(B, C, 32, 32, 32)
